# baseline (speedup 1.0000x reference)
import functools

import jax
import jax.numpy as jnp
from jax import lax
from jax.experimental import pallas as pl
from jax.experimental.pallas import tpu as pltpu
from jax.experimental.pallas import tpu_sc as plsc

N_NODES = 10000
N_EDGES = 320000
D_FEAT = 128
NC, NS, L = 2, 16, 16
NPAD = 10240
NB = NPAD // 128
R_TOT = 2560
ROWS = R_TOT // NS
R_REAL = N_EDGES // 128
NPS = NPAD // NS


def _mesh():
    return plsc.VectorSubcoreMesh(core_axis_name="c", subcore_axis_name="s")


def _rsqrt16(d):
    i = plsc.bitcast(d, jnp.int32)
    i = jnp.int32(0x5F3759DF) - lax.shift_right_logical(i, jnp.int32(1))
    y = plsc.bitcast(i, jnp.float32)
    for _ in range(3):
        y = y * (1.5 - 0.5 * d * y * y)
    return y


def _mv_body(x3_ref, w_ref, y_ref):
    y_ref[...] = jnp.sum(x3_ref[...] * w_ref[...][None], axis=2)


def _deg_body(ei3, zeros1, degp, dst_v, zb, sl_v, ones_v, w_sh, sem, sem_d):
    cid = lax.axis_index("c")
    sid = lax.axis_index("s")
    s0 = pl.multiple_of(sid * NPS, NPS)
    hrows = ROWS // 2
    rs = pl.multiple_of(cid * (R_TOT // 2) + sid * hrows, hrows)
    nrows = jnp.where(jnp.logical_and(cid == 1, sid == NS - 1),
                      R_REAL - (R_TOT // 2) - (NS - 1) * hrows, hrows)

    pltpu.async_copy(ei3.at[1, pl.ds(rs, hrows)], dst_v, sem_d)
    for c in range(128 // L):
        ones_v[pl.ds(c * L, L)] = jnp.full((L,), 1.0, jnp.float32)

    @pl.when(sid == 0)
    def _():
        pltpu.sync_copy(zeros1, zb)
        pltpu.sync_copy(zb, w_sh)

    plsc.subcore_barrier()
    pltpu.make_async_copy(ei3.at[1, pl.ds(rs, hrows)], dst_v, sem_d).wait()

    @plsc.parallel_loop(0, nrows, unroll=8)
    def _(j):
        pltpu.async_copy(ones_v, w_sh.at[dst_v.at[j]], sem, add=True)

    def drain(j, carry):
        pltpu.make_async_copy(ones_v, w_sh.at[dst_v.at[j]], sem).wait()
        return carry

    lax.fori_loop(0, nrows, drain, 0)
    plsc.subcore_barrier()
    pltpu.sync_copy(w_sh.at[pl.ds(s0, NPS)], sl_v)
    pltpu.sync_copy(sl_v, degp.at[cid, pl.ds(s0, NPS)])


def _sgc_body(ei3, y2, degp, b16, out,
              src_v, dst_v, vals_v, dis_v, sl_v, pb_v, yf_v, u_full, b_v,
              u_sh, t_sh, w_sh, sem, sem_s, sem_d, sem_y):
    cid = lax.axis_index("c")
    sid = lax.axis_index("s")
    s0 = pl.multiple_of(sid * NPS, NPS)
    rs = pl.multiple_of(sid * ROWS, ROWS)
    nrows = jnp.where(sid == NS - 1, R_REAL - (NS - 1) * ROWS, ROWS)

    with jax.named_scope("stage"):
        pltpu.async_copy(ei3.at[0, pl.ds(rs, ROWS)], src_v, sem_s)
        pltpu.async_copy(ei3.at[1, pl.ds(rs, ROWS)], dst_v, sem_d)
        pltpu.async_copy(y2, yf_v, sem_y)
        pltpu.sync_copy(b16, b_v)

    with jax.named_scope("build_u0"):
        pltpu.sync_copy(degp.at[0, pl.ds(s0, NPS)], sl_v)
        pltpu.sync_copy(degp.at[1, pl.ds(s0, NPS)], pb_v)
        pltpu.make_async_copy(y2, yf_v, sem_y).wait()
        r0 = sid * (NPS // 128)
        for k in range(NPS // L):
            sl = pl.ds(k * L, L)
            dis = _rsqrt16(sl_v[sl] + pb_v[sl] + 1.0)
            yk = yf_v[r0 + k // 8, pl.ds((k % 8) * L, L)]
            dis_v[sl] = dis
            sl_v[sl] = dis * yk
        pltpu.sync_copy(sl_v, u_sh.at[pl.ds(s0, NPS)])
        pltpu.sync_copy(sl_v, t_sh.at[pl.ds(s0, NPS)])
    plsc.subcore_barrier()

    with jax.named_scope("hop1"):
        pltpu.make_async_copy(ei3.at[0, pl.ds(rs, ROWS)], src_v, sem_s).wait()
        pltpu.make_async_copy(ei3.at[1, pl.ds(rs, ROWS)], dst_v, sem_d).wait()
        pltpu.sync_copy(u_sh, u_full)

        @plsc.parallel_loop(0, nrows, unroll=8)
        def _(j):
            for c in range(128 // L):
                sl = pl.ds(c * L, L)
                vals_v[j, sl] = plsc.load_gather(u_full, [src_v[j, sl]])
            pltpu.async_copy(vals_v.at[j], t_sh.at[dst_v.at[j]], sem,
                             add=True)

        def drain1(j, carry):
            pltpu.make_async_copy(vals_v.at[j], t_sh.at[dst_v.at[j]],
                                  sem).wait()
            return carry

        lax.fori_loop(0, nrows, drain1, 0)
    plsc.subcore_barrier()

    with jax.named_scope("build_u1"):
        pltpu.sync_copy(t_sh.at[pl.ds(s0, NPS)], sl_v)
        for k in range(NPS // L):
            sl = pl.ds(k * L, L)
            dis = dis_v[sl]
            sl_v[sl] = dis * dis * sl_v[sl]
        pltpu.sync_copy(sl_v, u_sh.at[pl.ds(s0, NPS)])
        pltpu.sync_copy(sl_v, w_sh.at[pl.ds(s0, NPS)])
    plsc.subcore_barrier()

    with jax.named_scope("hop2"):
        pltpu.sync_copy(u_sh, u_full)

        @plsc.parallel_loop(0, nrows, unroll=8)
        def _(j):
            for c in range(128 // L):
                sl = pl.ds(c * L, L)
                vals_v[j, sl] = plsc.load_gather(u_full, [src_v[j, sl]])
            pltpu.async_copy(vals_v.at[j], w_sh.at[dst_v.at[j]], sem,
                             add=True)

        def drain2(j, carry):
            pltpu.make_async_copy(vals_v.at[j], w_sh.at[dst_v.at[j]],
                                  sem).wait()
            return carry

        lax.fori_loop(0, nrows, drain2, 0)
    plsc.subcore_barrier()

    with jax.named_scope("epilogue"):
        @pl.when(cid == 0)
        def _():
            pltpu.sync_copy(w_sh.at[pl.ds(s0, NPS)], sl_v)
            for k in range(NPS // L):
                sl = pl.ds(k * L, L)
                h = dis_v[sl] * sl_v[sl] + b_v[...]
                sl_v[sl] = h * h
            pltpu.sync_copy(sl_v, out.at[pl.ds(s0, NPS)])


@functools.lru_cache(maxsize=None)
def _deg_kernel():
    return functools.partial(
        pl.kernel,
        mesh=_mesh(),
        compiler_params=pltpu.CompilerParams(needs_layout_passes=False),
        out_type=[jax.ShapeDtypeStruct((NC, NPAD), jnp.float32)],
        scratch_types=[
            pltpu.VMEM((ROWS // 2, 128), jnp.int32),
            pltpu.VMEM((NPAD,), jnp.float32),
            pltpu.VMEM((NPS,), jnp.float32),
            pltpu.VMEM((128,), jnp.float32),
            pltpu.VMEM_SHARED((NPAD,), jnp.float32),
            pltpu.SemaphoreType.DMA,
            pltpu.SemaphoreType.DMA,
        ],
    )(_deg_body)


@functools.lru_cache(maxsize=None)
def _sc_kernel():
    return functools.partial(
        pl.kernel,
        mesh=_mesh(),
        compiler_params=pltpu.CompilerParams(needs_layout_passes=False),
        out_type=[jax.ShapeDtypeStruct((NPAD,), jnp.float32)],
        scratch_types=[
            pltpu.VMEM((ROWS, 128), jnp.int32),
            pltpu.VMEM((ROWS, 128), jnp.int32),
            pltpu.VMEM((ROWS, 128), jnp.float32),
            pltpu.VMEM((NPS,), jnp.float32),
            pltpu.VMEM((NPS,), jnp.float32),
            pltpu.VMEM((NPS,), jnp.float32),
            pltpu.VMEM((NB, 128), jnp.float32),
            pltpu.VMEM((NPAD,), jnp.float32),
            pltpu.VMEM((L,), jnp.float32),
            pltpu.VMEM_SHARED((NPAD,), jnp.float32),
            pltpu.VMEM_SHARED((NPAD,), jnp.float32),
            pltpu.VMEM_SHARED((NPAD,), jnp.float32),
            pltpu.SemaphoreType.DMA,
            pltpu.SemaphoreType.DMA,
            pltpu.SemaphoreType.DMA,
            pltpu.SemaphoreType.DMA,
        ],
    )(_sgc_body)


def kernel(x, edge_index, W, b):
    ei3 = jnp.pad(edge_index.astype(jnp.int32),
                  ((0, 0), (0, R_TOT * 128 - N_EDGES))).reshape(2, R_TOT, 128)
    xpad = jnp.pad(x.astype(jnp.float32), ((0, NPAD - N_NODES), (0, 0)))
    x3 = xpad.reshape(NB, 128, D_FEAT)
    wrow = W.astype(jnp.float32).reshape(1, D_FEAT)
    b16 = jnp.broadcast_to(b.astype(jnp.float32).reshape(1), (L,))
    zeros1 = jnp.zeros((NPAD,), jnp.float32)

    y2 = pl.pallas_call(
        _mv_body,
        out_shape=jax.ShapeDtypeStruct((NB, 128), jnp.float32),
    )(x3, wrow)

    degp = _deg_kernel()(ei3, zeros1)
    if isinstance(degp, (tuple, list)):
        (degp,) = degp
    o = _sc_kernel()(ei3, y2, degp, b16)
    if isinstance(o, (tuple, list)):
        (o,) = o
    return o.reshape(NPAD, 1)[:N_NODES]
